# baseline (speedup 1.0000x reference)
import jax
import jax.numpy as jnp
from jax import lax
from jax.experimental import pallas as pl
from jax.experimental.pallas import tpu as pltpu
from jax.experimental.pallas import tpu_sc as plsc

B = 4096
L = 200
H = 64
V = 1000001
NC = 2
NS = 16
NL_BIG = 7
NL_SML = 6
SCH = 62464
STL = V - 15 * SCH
STLA = (STL // 128) * 128
TW = 80


def _body(tokT_hbm, tabT_hbm, posT_hbm, tail_hbm, out_hbm,
          row_sh, idxf, fb0, fb1, pos0, pos1, tailv,
          tsem, ssem, gsem, wsem, psem):
    c = lax.axis_index("c")
    s = lax.axis_index("s")
    t = c * NS + s
    l0 = jnp.where(t < 8, NL_BIG * t, 56 + NL_SML * (t - 8))
    fbufs = [fb0, fb1]
    posb = [pos0, pos1]

    def tok_stage(nl):
        for j in range(nl):
            pltpu.async_copy(tokT_hbm.at[l0 + j, pl.ds(0, B)],
                             idxf.at[pl.ds(j * B, B)], tsem)
        for j in range(nl):
            pltpu.make_async_copy(tokT_hbm.at[0, pl.ds(0, B)],
                                  idxf.at[pl.ds(0, B)], tsem).wait()

    @pl.when(t < 8)
    def _():
        tok_stage(NL_BIG)

    @pl.when(t >= 8)
    def _():
        tok_stage(NL_SML)

    soff = pl.multiple_of(s * SCH, 8)

    def stage_row(h):
        @pl.when(s < NS - 1)
        def _():
            pltpu.async_copy(tabT_hbm.at[h, pl.ds(soff, SCH)],
                             row_sh.at[pl.ds(soff, SCH)], ssem)
        @pl.when(s == NS - 1)
        def _():
            pltpu.async_copy(tabT_hbm.at[h, pl.ds(15 * SCH, STLA)],
                             row_sh.at[pl.ds(15 * SCH, STLA)], ssem)
            pltpu.async_copy(tail_hbm.at[pl.ds(h * TW, TW)], tailv, ssem)

    def wait_stage():
        @pl.when(s < NS - 1)
        def _():
            pltpu.make_async_copy(tabT_hbm.at[0, pl.ds(0, SCH)],
                                  row_sh.at[pl.ds(0, SCH)], ssem).wait()
        @pl.when(s == NS - 1)
        def _():
            pltpu.make_async_copy(tabT_hbm.at[0, pl.ds(15 * SCH, STLA)],
                                  row_sh.at[pl.ds(15 * SCH, STLA)],
                                  ssem).wait()
            pltpu.make_async_copy(tail_hbm.at[pl.ds(0, TW)],
                                  tailv, ssem).wait()
            pltpu.sync_copy(tailv, row_sh.at[pl.ds(15 * SCH + STLA, TW)])

    stage_row(0)
    pltpu.async_copy(posT_hbm.at[pl.ds(0, L)], pos0.at[pl.ds(0, L)], psem)

    def run_positions(h, posv, nl):
        for j in range(nl):
            fb = fbufs[j % 2]
            pltpu.make_async_copy(row_sh.at[idxf.at[pl.ds(0, B)]],
                                  fb, gsem).wait()
            if j + 1 < nl:
                if j >= 1:
                    pltpu.make_async_copy(
                        fbufs[(j + 1) % 2], out_hbm.at[0, 0, pl.ds(0, B)],
                        wsem).wait()
                pltpu.async_copy(
                    row_sh.at[idxf.at[pl.ds((j + 1) * B, B)]],
                    fbufs[(j + 1) % 2], gsem)
            else:
                plsc.subcore_barrier()

                @pl.when(h + 1 < H)
                def _():
                    stage_row(h + 1)

            spl = plsc.load_gather(
                posv, [jnp.zeros((16,), jnp.int32) + (l0 + j)])

            def add16(g, _):
                for u in range(16):
                    sl = pl.ds(g * 256 + u * 16, 16)
                    fb[sl] = fb[sl] + spl
                return 0

            lax.fori_loop(0, B // 256, add16, 0)

            pltpu.async_copy(
                fb, out_hbm.at[l0 + j, h, pl.ds(0, B)], wsem)

    def feature_pair(o, _):
      for hb in range(2):
        h = 2 * o + hb
        posv = posb[hb]

        wait_stage()
        plsc.subcore_barrier()

        pltpu.make_async_copy(posT_hbm.at[pl.ds(0, L)],
                              posv.at[pl.ds(0, L)], psem).wait()
        @pl.when(h + 1 < H)
        def _():
            pltpu.async_copy(posT_hbm.at[pl.ds((h + 1) * L, L)],
                             posb[1 - hb].at[pl.ds(0, L)], psem)

        @pl.when(h >= 1)
        def _():
            pltpu.make_async_copy(
                fb0, out_hbm.at[0, 0, pl.ds(0, B)], wsem).wait()
            pltpu.make_async_copy(
                fb1, out_hbm.at[0, 0, pl.ds(0, B)], wsem).wait()

        pltpu.async_copy(row_sh.at[idxf.at[pl.ds(0, B)]], fb0, gsem)

        @pl.when(t < 8)
        def _():
            run_positions(h, posv, NL_BIG)

        @pl.when(t >= 8)
        def _():
            run_positions(h, posv, NL_SML)
      return 0

    lax.fori_loop(0, H // 2, feature_pair, 0)

    pltpu.make_async_copy(fb0, out_hbm.at[0, 0, pl.ds(0, B)], wsem).wait()
    pltpu.make_async_copy(fb1, out_hbm.at[0, 0, pl.ds(0, B)], wsem).wait()


@jax.jit
def _encode(tokens_t, table_t, pos_t, tail_flat):
    mesh = plsc.VectorSubcoreMesh(core_axis_name="c", subcore_axis_name="s")
    return pl.kernel(
        _body,
        out_type=jax.ShapeDtypeStruct((L, H, B), jnp.float32),
        mesh=mesh,
        scratch_types=[
            pltpu.VMEM_SHARED((V + 15,), jnp.float32),
            pltpu.VMEM((NL_BIG * B,), jnp.int32),
            pltpu.VMEM((B,), jnp.float32),
            pltpu.VMEM((B,), jnp.float32),
            pltpu.VMEM((256,), jnp.float32),
            pltpu.VMEM((256,), jnp.float32),
            pltpu.VMEM((TW,), jnp.float32),
            pltpu.SemaphoreType.DMA,
            pltpu.SemaphoreType.DMA,
            pltpu.SemaphoreType.DMA,
            pltpu.SemaphoreType.DMA,
            pltpu.SemaphoreType.DMA,
        ],
        compiler_params=pltpu.CompilerParams(use_tc_tiling_on_sc=True,
                                            needs_layout_passes=False),
    )(tokens_t, table_t, pos_t, tail_flat)


def kernel(tokens, token_table, pos_table):
    tokens_t = tokens.astype(jnp.int32).T
    table_t = token_table.T
    pos_t = pos_table.T.reshape(-1)
    tail_flat = jnp.pad(token_table[15 * SCH + STLA:].T,
                        ((0, 0), (0, TW - (V - 15 * SCH - STLA)))).reshape(-1)
    out_t = _encode(tokens_t, table_t, pos_t, tail_flat)
    return jnp.transpose(out_t, (2, 0, 1))


# speedup vs baseline: 1.0905x; 1.0905x over previous
import jax
import jax.numpy as jnp
from jax import lax
from jax.experimental import pallas as pl
from jax.experimental.pallas import tpu as pltpu
from jax.experimental.pallas import tpu_sc as plsc

B = 4096
L = 200
H = 64
V = 1000001
NC = 2
NS = 16
BPT = B // (NC * NS)
NVAL = L * BPT
NQ = 4
LQ = L // NQ
QVAL = NVAL // NQ


def _body(tokT_hbm, tabT_hbm, posT_hbm, tail_hbm, out_hbm,
          row_sh, idxf, qb0, qb1, v2d, pos0, pos1, tailv,
          tsem, ssem, gsem, wsem, psem):
    c = lax.axis_index("c")
    s = lax.axis_index("s")
    b0 = pl.multiple_of((c * NS + s) * BPT, BPT)
    qbufs = [qb0, qb1]
    posb = [pos0, pos1]

    def tok_fire(l, _):
        pltpu.async_copy(tokT_hbm.at[l, pl.ds(b0, BPT)],
                         idxf.at[pl.ds(l * BPT, BPT)], tsem)
        return 0

    lax.fori_loop(0, L, tok_fire, 0)

    def tok_drain(l, _):
        pltpu.make_async_copy(tokT_hbm.at[0, pl.ds(b0, BPT)],
                              idxf.at[pl.ds(0, BPT)], tsem).wait()
        return 0

    lax.fori_loop(0, L, tok_drain, 0)

    SCH = 62464
    STL = V - 15 * SCH
    STLA = (STL // 128) * 128
    TW = 80
    soff = pl.multiple_of(s * SCH, 8)

    def stage_row(h):
        @pl.when(s < NS - 1)
        def _():
            pltpu.async_copy(tabT_hbm.at[h, pl.ds(soff, SCH)],
                             row_sh.at[pl.ds(soff, SCH)], ssem)
        @pl.when(s == NS - 1)
        def _():
            pltpu.async_copy(tabT_hbm.at[h, pl.ds(15 * SCH, STLA)],
                             row_sh.at[pl.ds(15 * SCH, STLA)], ssem)
            pltpu.async_copy(tail_hbm.at[pl.ds(h * TW, TW)], tailv, ssem)

    def wait_stage():
        @pl.when(s < NS - 1)
        def _():
            pltpu.make_async_copy(tabT_hbm.at[0, pl.ds(0, SCH)],
                                  row_sh.at[pl.ds(0, SCH)], ssem).wait()
        @pl.when(s == NS - 1)
        def _():
            pltpu.make_async_copy(tabT_hbm.at[0, pl.ds(15 * SCH, STLA)],
                                  row_sh.at[pl.ds(15 * SCH, STLA)],
                                  ssem).wait()
            pltpu.make_async_copy(tail_hbm.at[pl.ds(0, TW)],
                                  tailv, ssem).wait()
            pltpu.sync_copy(tailv, row_sh.at[pl.ds(15 * SCH + STLA, TW)])

    stage_row(0)
    pltpu.async_copy(posT_hbm.at[pl.ds(0, L)], pos0.at[pl.ds(0, L)], psem)

    def feature_pair(o, _):
      for hb in range(2):
        h = 2 * o + hb
        posv = posb[hb]

        wait_stage()
        plsc.subcore_barrier()

        pltpu.make_async_copy(posT_hbm.at[pl.ds(0, L)],
                              posv.at[pl.ds(0, L)], psem).wait()
        @pl.when(h + 1 < H)
        def _():
            pltpu.async_copy(posT_hbm.at[pl.ds((h + 1) * L, L)],
                             posb[1 - hb].at[pl.ds(0, L)], psem)

        @pl.when(h >= 1)
        def _():
            pltpu.make_async_copy(
                v2d, out_hbm.at[pl.ds(0, L), 0, pl.ds(b0, BPT)], wsem).wait()

        QH = QVAL // 2
        def g_issue(q, b):
            pltpu.async_copy(row_sh.at[idxf.at[pl.ds(q * QVAL, QH)]],
                             qbufs[b].at[pl.ds(0, QH)], gsem)
            pltpu.async_copy(row_sh.at[idxf.at[pl.ds(q * QVAL + QH, QH)]],
                             qbufs[b].at[pl.ds(QH, QH)], gsem)

        def g_wait(b):
            pltpu.make_async_copy(row_sh.at[idxf.at[pl.ds(0, QH)]],
                                  qbufs[b].at[pl.ds(0, QH)], gsem).wait()
            pltpu.make_async_copy(row_sh.at[idxf.at[pl.ds(0, QH)]],
                                  qbufs[b].at[pl.ds(QH, QH)], gsem).wait()

        g_issue(0, 0)
        for q in range(NQ):
            qb = qbufs[q % 2]
            g_wait(q % 2)
            if q + 1 < NQ:
                g_issue(q + 1, (q + 1) % 2)
            else:
                plsc.subcore_barrier()

                @pl.when(h + 1 < H)
                def _():
                    stage_row(h + 1)

            def add_group(g, _):
                pv16 = posv[pl.ds(q * LQ + 16 * g, 16)]
                for k in range(16):
                    i = 16 * g + k
                    l = q * LQ + i
                    spl = pv16[k] + jnp.zeros((16,), jnp.float32)
                    for cc in range(BPT // 16):
                        v2d[l, pl.ds(cc * 16, 16)] = (
                            qb[pl.ds(i * BPT + cc * 16, 16)] + spl)
                return 0

            lax.fori_loop(0, LQ // 16, add_group, 0)
            pvt = posv[pl.ds(q * LQ + LQ - 16, 16)]
            for k in range(14, 16):
                i = LQ - 16 + k
                l = q * LQ + i
                spl = pvt[k] + jnp.zeros((16,), jnp.float32)
                for cc in range(BPT // 16):
                    v2d[l, pl.ds(cc * 16, 16)] = (
                        qb[pl.ds(i * BPT + cc * 16, 16)] + spl)

        pltpu.async_copy(
            v2d, out_hbm.at[pl.ds(0, L), h, pl.ds(b0, BPT)], wsem)
      return 0

    lax.fori_loop(0, H // 2, feature_pair, 0)

    pltpu.make_async_copy(
        v2d, out_hbm.at[pl.ds(0, L), 0, pl.ds(b0, BPT)], wsem).wait()


@jax.jit
def _encode(tokens_t, table_t, pos_t, tail_flat):
    mesh = plsc.VectorSubcoreMesh(core_axis_name="c", subcore_axis_name="s")
    return pl.kernel(
        _body,
        out_type=jax.ShapeDtypeStruct((L, H, B), jnp.float32),
        mesh=mesh,
        scratch_types=[
            pltpu.VMEM_SHARED((V + 15,), jnp.float32),
            pltpu.VMEM((NVAL,), jnp.int32),
            pltpu.VMEM((QVAL,), jnp.float32),
            pltpu.VMEM((QVAL,), jnp.float32),
            pltpu.VMEM((L, BPT), jnp.float32),
            pltpu.VMEM((256,), jnp.float32),
            pltpu.VMEM((256,), jnp.float32),
            pltpu.VMEM((80,), jnp.float32),
            pltpu.SemaphoreType.DMA,
            pltpu.SemaphoreType.DMA,
            pltpu.SemaphoreType.DMA,
            pltpu.SemaphoreType.DMA,
            pltpu.SemaphoreType.DMA,
        ],
        compiler_params=pltpu.CompilerParams(use_tc_tiling_on_sc=True),
    )(tokens_t, table_t, pos_t, tail_flat)


def kernel(tokens, token_table, pos_table):
    tokens_t = tokens.astype(jnp.int32).T
    table_t = token_table.T
    pos_t = pos_table.T.reshape(-1)
    tail_flat = jnp.pad(token_table[15 * 62464 + 62976:].T,
                        ((0, 0), (0, 15))).reshape(-1)
    out_t = _encode(tokens_t, table_t, pos_t, tail_flat)
    return jnp.transpose(out_t, (2, 0, 1))
